# baseline (speedup 1.0000x reference)
import functools
import math

import jax
import jax.numpy as jnp
from jax import lax
from jax.experimental import pallas as pl
from jax.experimental.pallas import tpu as pltpu
from jax.experimental.pallas import tpu_sc as plsc

VOCAB_SIZE = 100000
D_MODEL = 1024
SCALE = math.sqrt(D_MODEL)

NC = 2
NS = 16
NW = NC * NS
LANES = 16

B_TOTAL = 4 * 4096
B_PER_W = B_TOTAL // NW
CHUNK = 4
N_CHUNKS = B_PER_W // CHUNK
NBUF = 8
N_WAVES = N_CHUNKS // NBUF
SLICES_PER_ROW = D_MODEL // LANES


def _scale_chunk(buf):
    @plsc.parallel_loop(0, CHUNK)
    def scale_row(r):
        @plsc.parallel_loop(0, SLICES_PER_ROW, unroll=8)
        def scale_slice(c):
            sl = pl.ds(c * LANES, LANES)
            buf[r, sl] = buf[r, sl] * SCALE


def _emb_kernel(x_hbm, table_hbm, out_hbm, idx_v, rows_v, in_sems, out_sems):
    wid = lax.axis_index("s") * NC + lax.axis_index("c")
    base = wid * B_PER_W

    pltpu.sync_copy(x_hbm.at[wid], idx_v)

    dummy_hbm = out_hbm.at[pl.ds(0, CHUNK)]

    def start_gather(g, b):
        pltpu.async_copy(table_hbm.at[idx_v.at[g]], rows_v.at[b],
                         in_sems.at[b])

    def wait_gather(b):
        pltpu.make_async_copy(dummy_hbm, rows_v.at[b], in_sems.at[b]).wait()

    def start_out(g, b):
        pltpu.async_copy(rows_v.at[b],
                         out_hbm.at[pl.ds(base + g * CHUNK, CHUNK)],
                         out_sems.at[b])

    def wait_out(b):
        pltpu.make_async_copy(rows_v.at[b], dummy_hbm, out_sems.at[b]).wait()

    for b in range(NBUF):
        start_gather(b, b)

    def wave(T, carry):
        t = T * NBUF
        for b in range(NBUF):
            wait_gather(b)
            _scale_chunk(rows_v.at[b])
            start_out(t + b, b)

            if b >= 1:
                @pl.when(T < N_WAVES - 1)
                def _():
                    wait_out(b - 1)
                    start_gather(t + NBUF + b - 1, b - 1)

        @pl.when(T < N_WAVES - 1)
        def _():
            wait_out(NBUF - 1)
            start_gather(t + 2 * NBUF - 1, NBUF - 1)

        return carry

    lax.fori_loop(0, N_WAVES, wave, None)

    for b in range(NBUF):
        wait_out(b)


@jax.jit
def _emb(x_flat, table):
    mesh = plsc.VectorSubcoreMesh(core_axis_name="c", subcore_axis_name="s")
    f = functools.partial(
        pl.kernel,
        mesh=mesh,
        out_type=jax.ShapeDtypeStruct((B_TOTAL, D_MODEL), jnp.float32),
        scratch_types=[
            pltpu.VMEM((N_CHUNKS, CHUNK), jnp.int32),
            pltpu.VMEM((NBUF, CHUNK, D_MODEL), jnp.float32),
            pltpu.SemaphoreType.DMA((NBUF,)),
            pltpu.SemaphoreType.DMA((NBUF,)),
        ],
    )(_emb_kernel)
    return f(x_flat.reshape(NW, N_CHUNKS, CHUNK), table)


def kernel(x, table):
    out = _emb(x.reshape(-1).astype(jnp.int32), table)
    return out.reshape(x.shape[0], x.shape[1], D_MODEL)


# speedup vs baseline: 1.0403x; 1.0403x over previous
import functools
import math

import jax
import jax.numpy as jnp
from jax import lax
from jax.experimental import pallas as pl
from jax.experimental.pallas import tpu as pltpu
from jax.experimental.pallas import tpu_sc as plsc

VOCAB_SIZE = 100000
D_MODEL = 1024
SCALE = math.sqrt(D_MODEL)

NC = 2
NS = 16
NW = NC * NS
LANES = 16

B_TOTAL = 4 * 4096
B_PER_W = B_TOTAL // NW
CHUNK = 8
N_CHUNKS = B_PER_W // CHUNK
NBUF = 8
N_WAVES = N_CHUNKS // NBUF
SLICES_PER_ROW = D_MODEL // LANES


def _scale_chunk(buf):
    @plsc.parallel_loop(0, CHUNK)
    def scale_row(r):
        @plsc.parallel_loop(0, SLICES_PER_ROW, unroll=8)
        def scale_slice(c):
            sl = pl.ds(c * LANES, LANES)
            buf[r, sl] = buf[r, sl] * SCALE


def _emb_kernel(x_hbm, table_hbm, out_hbm, idx_v, rows_v, in_sems, out_sems):
    wid = lax.axis_index("s") * NC + lax.axis_index("c")
    base = wid * B_PER_W

    pltpu.sync_copy(x_hbm.at[wid], idx_v)

    dummy_hbm = out_hbm.at[pl.ds(0, CHUNK)]

    def start_gather(g, b):
        pltpu.async_copy(table_hbm.at[idx_v.at[g]], rows_v.at[b],
                         in_sems.at[b])

    def wait_gather(b):
        pltpu.make_async_copy(dummy_hbm, rows_v.at[b], in_sems.at[b]).wait()

    def start_out(g, b):
        pltpu.async_copy(rows_v.at[b],
                         out_hbm.at[pl.ds(base + g * CHUNK, CHUNK)],
                         out_sems.at[b])

    def wait_out(b):
        pltpu.make_async_copy(rows_v.at[b], dummy_hbm, out_sems.at[b]).wait()

    for b in range(NBUF):
        start_gather(b, b)

    def wave(T, carry):
        t = T * NBUF
        for b in range(NBUF):
            wait_gather(b)
            _scale_chunk(rows_v.at[b])
            start_out(t + b, b)

            if b >= 1:
                @pl.when(T < N_WAVES - 1)
                def _():
                    wait_out(b - 1)
                    start_gather(t + NBUF + b - 1, b - 1)

        @pl.when(T < N_WAVES - 1)
        def _():
            wait_out(NBUF - 1)
            start_gather(t + 2 * NBUF - 1, NBUF - 1)

        return carry

    lax.fori_loop(0, N_WAVES, wave, None)

    for b in range(NBUF):
        wait_out(b)


@jax.jit
def _emb(x_flat, table):
    mesh = plsc.VectorSubcoreMesh(core_axis_name="c", subcore_axis_name="s")
    f = functools.partial(
        pl.kernel,
        mesh=mesh,
        out_type=jax.ShapeDtypeStruct((B_TOTAL, D_MODEL), jnp.float32),
        scratch_types=[
            pltpu.VMEM((N_CHUNKS, CHUNK), jnp.int32),
            pltpu.VMEM((NBUF, CHUNK, D_MODEL), jnp.float32),
            pltpu.SemaphoreType.DMA((NBUF,)),
            pltpu.SemaphoreType.DMA((NBUF,)),
        ],
    )(_emb_kernel)
    return f(x_flat.reshape(NW, N_CHUNKS, CHUNK), table)


def kernel(x, table):
    out = _emb(x.reshape(-1).astype(jnp.int32), table)
    return out.reshape(x.shape[0], x.shape[1], D_MODEL)
